# baseline (speedup 1.0000x reference)
import functools

import jax
import jax.numpy as jnp
from jax import lax
from jax.experimental import pallas as pl
from jax.experimental.pallas import tpu as pltpu
from jax.experimental.pallas import tpu_sc as plsc

N_NODES = 10000
N_EDGES = 320000
D = 128
D_EDGE = 16
EPS = 1e-5

NB = 32
SEG = 313
NPAD = NB * SEG
EPW = N_EDGES // NB
CH = 80
NCH = EPW // CH
SC_CH = 1600
N_OUT = N_EDGES // SC_CH
QL = 256
RB = 128
NBK = 64
SEGK = 157
NPADK = NBK * SEGK
FMIN = -3.0e38
FMAX = 3.0e38

_SC_PARAMS = pltpu.CompilerParams(needs_layout_passes=False)


def _ab_body(h_ref, w_ref, a_ref, b_ref):
    hb = h_ref[...]
    a_ref[...] = jnp.dot(hb, w_ref[0:D, :], preferred_element_type=jnp.float32)
    b_ref[...] = jnp.dot(hb, w_ref[D:2 * D, :], preferred_element_type=jnp.float32)


def _compute_ab(h, W_pre):
    blk = 1000
    grid = N_NODES // blk
    return pl.pallas_call(
        _ab_body,
        grid=(grid,),
        in_specs=[
            pl.BlockSpec((blk, D), lambda i: (i, 0)),
            pl.BlockSpec((2 * D + D_EDGE, D), lambda i: (0, 0)),
        ],
        out_specs=[
            pl.BlockSpec((blk, D), lambda i: (i, 0)),
            pl.BlockSpec((blk, D), lambda i: (i, 0)),
        ],
        out_shape=[
            jax.ShapeDtypeStruct((N_NODES, D), jnp.float32),
            jax.ShapeDtypeStruct((N_NODES, D), jnp.float32),
        ],
    )(h, W_pre)


def _c_body(e_ref, w_ref, bp_ref, c_ref):
    c_ref[...] = (
        jnp.dot(e_ref[...], w_ref[2 * D:2 * D + D_EDGE, :],
                preferred_element_type=jnp.float32)
        + bp_ref[...]
    )


def _compute_c(e, W_pre, b_pre2):
    blk = 4000
    grid = N_EDGES // blk
    return pl.pallas_call(
        _c_body,
        grid=(grid,),
        in_specs=[
            pl.BlockSpec((blk, D_EDGE), lambda i: (i, 0)),
            pl.BlockSpec((2 * D + D_EDGE, D), lambda i: (0, 0)),
            pl.BlockSpec((1, D), lambda i: (0, 0)),
        ],
        out_specs=pl.BlockSpec((blk, D), lambda i: (i, 0)),
        out_shape=jax.ShapeDtypeStruct((N_EDGES, D), jnp.float32),
    )(e, W_pre, b_pre2)


def _m_builder(A, C, src):
    mesh = plsc.VectorSubcoreMesh(core_axis_name="c", subcore_axis_name="s")

    @functools.partial(
        pl.kernel,
        out_type=jax.ShapeDtypeStruct((N_EDGES, D), jnp.float32),
        mesh=mesh,
        compiler_params=_SC_PARAMS,
        scratch_types=[
            pltpu.VMEM((EPW,), jnp.int32),
            pltpu.VMEM((CH, D), jnp.float32),
            pltpu.VMEM((CH, D), jnp.float32),
            pltpu.VMEM((CH, D), jnp.float32),
            pltpu.VMEM((CH, D), jnp.float32),
            pltpu.VMEM((CH, D), jnp.float32),
            pltpu.VMEM((CH, D), jnp.float32),
            pltpu.SemaphoreType.DMA,
            pltpu.SemaphoreType.DMA,
            pltpu.SemaphoreType.DMA,
            pltpu.SemaphoreType.DMA,
            pltpu.SemaphoreType.DMA,
            pltpu.SemaphoreType.DMA,
        ],
    )
    def k(a_hbm, c_hbm, src_hbm, m_hbm, srcall, ab0, ab1, cb0, cb1, mb0, mb1,
          sg0, sg1, sc0, sc1, sw0, sw1):
        t = lax.axis_index("s") * 2 + lax.axis_index("c")
        tbase = t * EPW
        pltpu.sync_copy(src_hbm.at[pl.ds(tbase, EPW)], srcall)

        def issue(i, ab, cb, sg, sc):
            pltpu.async_copy(a_hbm.at[srcall.at[pl.ds(i * CH, CH)]], ab, sg)
            pltpu.async_copy(c_hbm.at[pl.ds(tbase + i * CH, CH), :], cb, sc)

        def adds(ab, cb, mb):
            def addrow(r, _):
                for f in range(D // 16):
                    sl = pl.ds(f * 16, 16)
                    mb[r, sl] = ab[r, sl] + cb[r, sl]
                return 0
            lax.fori_loop(0, CH, addrow, 0)

        def gwait(i, ab, cb, sg, sc):
            pltpu.make_async_copy(
                a_hbm.at[srcall.at[pl.ds(i * CH, CH)]], ab, sg).wait()
            pltpu.make_async_copy(
                c_hbm.at[pl.ds(tbase + i * CH, CH), :], cb, sc).wait()

        def wwait(i, mb, sw):
            pltpu.make_async_copy(
                mb, m_hbm.at[pl.ds(tbase + i * CH, CH), :], sw).wait()

        issue(0, ab0, cb0, sg0, sc0)
        issue(1, ab1, cb1, sg1, sc1)

        def pair(pi, _):
            i0 = 2 * pi
            gwait(i0, ab0, cb0, sg0, sc0)

            @pl.when(pi > 0)
            def _():
                wwait(i0 - 2, mb0, sw0)

            adds(ab0, cb0, mb0)
            pltpu.async_copy(mb0, m_hbm.at[pl.ds(tbase + i0 * CH, CH), :], sw0)

            @pl.when(i0 + 2 < NCH)
            def _():
                issue(i0 + 2, ab0, cb0, sg0, sc0)

            i1 = i0 + 1
            gwait(i1, ab1, cb1, sg1, sc1)

            @pl.when(pi > 0)
            def _():
                wwait(i1 - 2, mb1, sw1)

            adds(ab1, cb1, mb1)
            pltpu.async_copy(mb1, m_hbm.at[pl.ds(tbase + i1 * CH, CH), :], sw1)

            @pl.when(i1 + 2 < NCH)
            def _():
                issue(i1 + 2, ab1, cb1, sg1, sc1)

            return 0

        lax.fori_loop(0, NCH // 2, pair, 0)

        last = NCH - 1
        gwait(last, ab0, cb0, sg0, sc0)
        wwait(last - 2, mb0, sw0)
        adds(ab0, cb0, mb0)
        pltpu.async_copy(mb0, m_hbm.at[pl.ds(tbase + last * CH, CH), :], sw0)
        wwait(last - 1, mb1, sw1)
        wwait(last, mb0, sw0)

    return k(A, C, src)


def _segment_pass(M, dst):
    mesh = plsc.VectorSubcoreMesh(core_axis_name="c", subcore_axis_name="s")

    @functools.partial(
        pl.kernel,
        out_type=[
            jax.ShapeDtypeStruct((NBK, SEGK + 1, D), jnp.float32),
            jax.ShapeDtypeStruct((NBK, SEGK + 1, D), jnp.float32),
            jax.ShapeDtypeStruct((NBK, SEGK + 1, D), jnp.float32),
            jax.ShapeDtypeStruct((NBK, SEGK + 1, 16), jnp.float32),
        ],
        mesh=mesh,
        compiler_params=_SC_PARAMS,
        scratch_types=[
            pltpu.VMEM((SEGK + 1, D), jnp.float32),
            pltpu.VMEM((SEGK + 1, D), jnp.float32),
            pltpu.VMEM((SEGK + 1, D), jnp.float32),
            pltpu.VMEM((SEGK + 1, 16), jnp.float32),
            pltpu.VMEM((SC_CH,), jnp.int32),
            pltpu.VMEM((SC_CH,), jnp.int32),
            pltpu.VMEM((QL // 8, 128), jnp.int32),
            pltpu.VMEM((QL // 8, 128), jnp.int32),
            pltpu.VMEM((2, RB), jnp.int32),
            pltpu.VMEM((2, RB), jnp.int32),
            pltpu.VMEM((2 * RB, D), jnp.float32),
            pltpu.SemaphoreType.DMA,
            pltpu.SemaphoreType.DMA,
            pltpu.SemaphoreType.DMA,
            pltpu.SemaphoreType.DMA,
        ],
    )
    def k(m_hbm, dst_hbm, sum_hbm, max_hbm, min_hbm, deg_hbm,
          sacc, xacc, nacc, dacc, dbuf0, dbuf1, eq2, dq2, stg, dls, mbuf,
          semg0, semg1, semd0, semd1):
        t = lax.axis_index("s") * 2 + lax.axis_index("c")
        iota = lax.broadcasted_iota(jnp.int32, (16,), 0)
        one0 = jnp.where(iota == 0, 1.0, 0.0)
        zero16 = jnp.zeros((16,), jnp.float32)
        zi = jnp.zeros((16,), jnp.int32)

        def phase(p):
            bkt = 2 * t + p
            lo = bkt * SEGK
            hi = lo + SEGK

            def init(i, _):
                for f in range(D // 16):
                    sl = pl.ds(f * 16, 16)
                    sacc[i, sl] = zero16
                    xacc[i, sl] = zero16 + FMIN
                    nacc[i, sl] = zero16 + FMAX
                dacc[i, pl.ds(0, 16)] = zero16
                return 0

            lax.fori_loop(0, SEGK + 1, init, 0)

            def form_issue(rc, prv, cnt):
                slot = rc & 1
                backlog = cnt - prv
                for r in range(RB // 16):
                    valid = backlog > r
                    pos = prv + r
                    rowr = (pos >> 3) & (QL // 8 - 1)
                    colr = iota * 8 + (pos & 7)
                    dlv = plsc.load_gather(dq2, [rowr, colr])
                    ev = plsc.load_gather(eq2, [rowr, colr])
                    dlv = jnp.where(valid, dlv, SEGK)
                    ev = jnp.where(valid, ev, 0)
                    stg[slot, pl.ds(r * 16, 16)] = ev
                    dls[slot, pl.ds(r * 16, 16)] = dlv

                @pl.when(slot == 0)
                def _():
                    pltpu.async_copy(
                        m_hbm.at[stg.at[0]], mbuf.at[pl.ds(0, RB), :], semg0)

                @pl.when(slot == 1)
                def _():
                    pltpu.async_copy(
                        m_hbm.at[stg.at[1]], mbuf.at[pl.ds(RB, RB), :], semg1)

                return prv + jnp.minimum(backlog, RB // 16)

            def process(rc):
                slot = rc & 1

                @pl.when(slot == 0)
                def _():
                    pltpu.make_async_copy(
                        m_hbm.at[stg.at[0]], mbuf.at[pl.ds(0, RB), :],
                        semg0).wait()

                @pl.when(slot == 1)
                def _():
                    pltpu.make_async_copy(
                        m_hbm.at[stg.at[1]], mbuf.at[pl.ds(RB, RB), :],
                        semg1).wait()

                def rmw(r, _):
                    dlv = dls[slot, pl.ds(r * 16, 16)]
                    for j in range(16):
                        dlj = jnp.sum(jnp.where(iota == j, dlv, 0))
                        row = slot * RB + r * 16 + j
                        for f in range(D // 16):
                            sl = pl.ds(f * 16, 16)
                            mrow = mbuf[row, sl]
                            sacc[dlj, sl] = sacc[dlj, sl] + mrow
                            xacc[dlj, sl] = jnp.maximum(xacc[dlj, sl], mrow)
                            nacc[dlj, sl] = jnp.minimum(nacc[dlj, sl], mrow)
                        dacc[dlj, pl.ds(0, 16)] = (
                            dacc[dlj, pl.ds(0, 16)] + one0)
                    return 0

                lax.fori_loop(0, RB // 16, rmw, 0)

            def step(c):
                prv, cnt, rc = c
                prv = form_issue(rc, prv, cnt)

                @pl.when(rc > 0)
                def _():
                    process(rc - 1)

                return (prv, cnt, rc + 1)

            def scan_chunk(dbuf, oi, carry):
                def vec(ii, c):
                    prv, cnt, rc = c
                    kv = dbuf[pl.ds(ii * 16, 16)]
                    m = (kv >= lo) & (kv < hi)
                    dl = kv - lo
                    eid = oi * SC_CH + ii * 16 + iota
                    rows = (cnt >> 3) & (QL // 8 - 1)
                    cols = iota * 8 + (cnt & 7)
                    plsc.store_scatter(dq2, [rows, cols], dl, mask=m)
                    plsc.store_scatter(eq2, [rows, cols], eid, mask=m)
                    return (prv, cnt + jnp.where(m, 1, 0), rc)

                carry = lax.fori_loop(0, SC_CH // 16, vec, carry)
                return lax.while_loop(
                    lambda c: jnp.sum(c[1] - c[0]) >= RB, step, carry)

            def pair(i, carry):
                pltpu.make_async_copy(
                    dst_hbm.at[pl.ds((2 * i) * SC_CH, SC_CH)], dbuf0,
                    semd0).wait()
                pltpu.async_copy(
                    dst_hbm.at[pl.ds((2 * i + 1) * SC_CH, SC_CH)], dbuf1,
                    semd1)
                carry = scan_chunk(dbuf0, 2 * i, carry)
                pltpu.make_async_copy(
                    dst_hbm.at[pl.ds((2 * i + 1) * SC_CH, SC_CH)], dbuf1,
                    semd1).wait()

                @pl.when(2 * i + 2 < N_OUT)
                def _():
                    pltpu.async_copy(
                        dst_hbm.at[pl.ds((2 * i + 2) * SC_CH, SC_CH)], dbuf0,
                        semd0)

                return scan_chunk(dbuf1, 2 * i + 1, carry)

            pltpu.async_copy(dst_hbm.at[pl.ds(0, SC_CH)], dbuf0, semd0)
            carry = lax.fori_loop(0, N_OUT // 2, pair, (zi, zi, 0))

            carry = lax.while_loop(
                lambda c: jnp.sum(c[1] - c[0]) > 0, step, carry)
            prv, cnt, rc = carry

            @pl.when(rc > 0)
            def _():
                process(rc - 1)

            pltpu.sync_copy(sacc, sum_hbm.at[bkt])
            pltpu.sync_copy(xacc, max_hbm.at[bkt])
            pltpu.sync_copy(nacc, min_hbm.at[bkt])
            pltpu.sync_copy(dacc, deg_hbm.at[bkt])

        phase(0)
        phase(1)

    return k(M, dst)


def _post_body(h_ref, bmat_ref, sum_ref, mx_ref, mn_ref, deg_ref, sn_ref,
               wp_ref, bp_ref, out_ref, stat_ref):
    i = pl.program_id(0)
    deg = deg_ref[...]
    pos = deg > 0.0
    invd = 1.0 / jnp.maximum(deg, 1.0)
    bmat = bmat_ref[...]
    mask = jnp.where(pos, 1.0, 0.0)
    mean = sum_ref[...] * invd + bmat * mask
    mx = jnp.where(pos, mx_ref[...] + bmat, 0.0)
    mn = jnp.where(pos, mn_ref[...] + bmat, 0.0)
    o = (
        jnp.dot(h_ref[...], wp_ref[0:D, :], preferred_element_type=jnp.float32)
        + jnp.dot(mean, wp_ref[D:2 * D, :], preferred_element_type=jnp.float32)
        + jnp.dot(mx, wp_ref[2 * D:3 * D, :], preferred_element_type=jnp.float32)
        + jnp.dot(mn, wp_ref[3 * D:4 * D, :], preferred_element_type=jnp.float32)
        + bp_ref[...]
    )
    o = o * sn_ref[...]
    out_ref[...] = o

    @pl.when(i == 0)
    def _():
        stat_ref[...] = jnp.zeros_like(stat_ref)

    s = jnp.sum(o, axis=0, keepdims=True)
    sq = jnp.sum(o * o, axis=0, keepdims=True)
    lane = lax.broadcasted_iota(jnp.int32, (8, D), 0)
    upd = jnp.where(lane == 0, s, 0.0) + jnp.where(lane == 1, sq, 0.0)
    stat_ref[...] = stat_ref[...] + upd


def _compute_post(h, bmat, sum_f, mx_f, mn_f, deg, snorm, W_post, b_post2):
    blk = 1000
    grid = N_NODES // blk
    return pl.pallas_call(
        _post_body,
        grid=(grid,),
        in_specs=[
            pl.BlockSpec((blk, D), lambda i: (i, 0)),
            pl.BlockSpec((blk, D), lambda i: (i, 0)),
            pl.BlockSpec((blk, D), lambda i: (i, 0)),
            pl.BlockSpec((blk, D), lambda i: (i, 0)),
            pl.BlockSpec((blk, D), lambda i: (i, 0)),
            pl.BlockSpec((blk, 1), lambda i: (i, 0)),
            pl.BlockSpec((blk, 1), lambda i: (i, 0)),
            pl.BlockSpec((4 * D, D), lambda i: (0, 0)),
            pl.BlockSpec((1, D), lambda i: (0, 0)),
        ],
        out_specs=[
            pl.BlockSpec((blk, D), lambda i: (i, 0)),
            pl.BlockSpec((8, D), lambda i: (0, 0)),
        ],
        out_shape=[
            jax.ShapeDtypeStruct((N_NODES, D), jnp.float32),
            jax.ShapeDtypeStruct((8, D), jnp.float32),
        ],
    )(h, bmat, sum_f, mx_f, mn_f, deg, snorm, W_post, b_post2)


def _bn_body(o_ref, stat_ref, g_ref, b_ref, out_ref):
    st = stat_ref[...]
    mu = st[0:1, :] / N_NODES
    var = st[1:2, :] / N_NODES - mu * mu
    inv = lax.rsqrt(var + EPS)
    out_ref[...] = (o_ref[...] - mu) * inv * g_ref[...] + b_ref[...]


def _compute_bn(out_pre, stats, gamma2, beta2):
    blk = 1000
    grid = N_NODES // blk
    return pl.pallas_call(
        _bn_body,
        grid=(grid,),
        in_specs=[
            pl.BlockSpec((blk, D), lambda i: (i, 0)),
            pl.BlockSpec((8, D), lambda i: (0, 0)),
            pl.BlockSpec((1, D), lambda i: (0, 0)),
            pl.BlockSpec((1, D), lambda i: (0, 0)),
        ],
        out_specs=pl.BlockSpec((blk, D), lambda i: (i, 0)),
        out_shape=jax.ShapeDtypeStruct((N_NODES, D), jnp.float32),
    )(out_pre, stats, gamma2, beta2)


def kernel(h, edge_index, e, snorm_n, eig, W_pre, b_pre, W_post, b_post,
           bn_gamma, bn_beta):
    src = edge_index[0]
    dst = edge_index[1]
    b_pre2 = b_pre.reshape(1, D)
    b_post2 = b_post.reshape(1, D)
    gamma2 = bn_gamma.reshape(1, D)
    beta2 = bn_beta.reshape(1, D)

    A, B = _compute_ab(h, W_pre)
    C = _compute_c(e, W_pre, b_pre2)
    M = _m_builder(A, C, src)
    sum3, mx3, mn3, deg2 = _segment_pass(M, dst)

    sum_f = sum3[:, :SEGK, :].reshape(NPADK, D)[:N_NODES]
    mx_f = mx3[:, :SEGK, :].reshape(NPADK, D)[:N_NODES]
    mn_f = mn3[:, :SEGK, :].reshape(NPADK, D)[:N_NODES]
    deg = deg2[:, :SEGK, 0].reshape(NPADK)[:N_NODES].reshape(N_NODES, 1)

    out_pre, stats = _compute_post(
        h, B, sum_f, mx_f, mn_f, deg, snorm_n, W_post, b_post2)
    return _compute_bn(out_pre, stats, gamma2, beta2)


# speedup vs baseline: 1.0642x; 1.0642x over previous
import functools

import jax
import jax.numpy as jnp
from jax import lax
from jax.experimental import pallas as pl
from jax.experimental.pallas import tpu as pltpu
from jax.experimental.pallas import tpu_sc as plsc

N_NODES = 10000
N_EDGES = 320000
D = 128
D_EDGE = 16
EPS = 1e-5

NB = 32
SEG = 313
NPAD = NB * SEG
EPW = N_EDGES // NB
CH = 80
NCH = EPW // CH
SC_CH = 3200
N_OUT = N_EDGES // SC_CH
QL = 512
RB = 128
NBK = 64
SEGK = 157
NPADK = NBK * SEGK
FMIN = -3.0e38
FMAX = 3.0e38

_SC_PARAMS = pltpu.CompilerParams(needs_layout_passes=False)


def _ab_body(h_ref, w_ref, a_ref, b_ref):
    hb = h_ref[...]
    a_ref[...] = jnp.dot(hb, w_ref[0:D, :], preferred_element_type=jnp.float32)
    b_ref[...] = jnp.dot(hb, w_ref[D:2 * D, :], preferred_element_type=jnp.float32)


def _compute_ab(h, W_pre):
    blk = 1000
    grid = N_NODES // blk
    return pl.pallas_call(
        _ab_body,
        grid=(grid,),
        in_specs=[
            pl.BlockSpec((blk, D), lambda i: (i, 0)),
            pl.BlockSpec((2 * D + D_EDGE, D), lambda i: (0, 0)),
        ],
        out_specs=[
            pl.BlockSpec((blk, D), lambda i: (i, 0)),
            pl.BlockSpec((blk, D), lambda i: (i, 0)),
        ],
        out_shape=[
            jax.ShapeDtypeStruct((N_NODES, D), jnp.float32),
            jax.ShapeDtypeStruct((N_NODES, D), jnp.float32),
        ],
    )(h, W_pre)


def _c_body(e_ref, w_ref, bp_ref, c_ref):
    c_ref[...] = (
        jnp.dot(e_ref[...], w_ref[2 * D:2 * D + D_EDGE, :],
                preferred_element_type=jnp.float32)
        + bp_ref[...]
    )


def _compute_c(e, W_pre, b_pre2):
    blk = 4000
    grid = N_EDGES // blk
    return pl.pallas_call(
        _c_body,
        grid=(grid,),
        in_specs=[
            pl.BlockSpec((blk, D_EDGE), lambda i: (i, 0)),
            pl.BlockSpec((2 * D + D_EDGE, D), lambda i: (0, 0)),
            pl.BlockSpec((1, D), lambda i: (0, 0)),
        ],
        out_specs=pl.BlockSpec((blk, D), lambda i: (i, 0)),
        out_shape=jax.ShapeDtypeStruct((N_EDGES, D), jnp.float32),
    )(e, W_pre, b_pre2)


def _m_builder(A, C, src):
    mesh = plsc.VectorSubcoreMesh(core_axis_name="c", subcore_axis_name="s")

    @functools.partial(
        pl.kernel,
        out_type=jax.ShapeDtypeStruct((N_EDGES, D), jnp.float32),
        mesh=mesh,
        compiler_params=_SC_PARAMS,
        scratch_types=[
            pltpu.VMEM((EPW,), jnp.int32),
            pltpu.VMEM((CH, D), jnp.float32),
            pltpu.VMEM((CH, D), jnp.float32),
            pltpu.VMEM((CH, D), jnp.float32),
            pltpu.VMEM((CH, D), jnp.float32),
            pltpu.VMEM((CH, D), jnp.float32),
            pltpu.VMEM((CH, D), jnp.float32),
            pltpu.SemaphoreType.DMA,
            pltpu.SemaphoreType.DMA,
            pltpu.SemaphoreType.DMA,
            pltpu.SemaphoreType.DMA,
            pltpu.SemaphoreType.DMA,
            pltpu.SemaphoreType.DMA,
        ],
    )
    def k(a_hbm, c_hbm, src_hbm, m_hbm, srcall, ab0, ab1, cb0, cb1, mb0, mb1,
          sg0, sg1, sc0, sc1, sw0, sw1):
        t = lax.axis_index("s") * 2 + lax.axis_index("c")
        tbase = t * EPW
        pltpu.sync_copy(src_hbm.at[pl.ds(tbase, EPW)], srcall)

        def issue(i, ab, cb, sg, sc):
            pltpu.async_copy(a_hbm.at[srcall.at[pl.ds(i * CH, CH)]], ab, sg)
            pltpu.async_copy(c_hbm.at[pl.ds(tbase + i * CH, CH), :], cb, sc)

        def adds(ab, cb, mb):
            def addrow(r, _):
                for f in range(D // 16):
                    sl = pl.ds(f * 16, 16)
                    mb[r, sl] = ab[r, sl] + cb[r, sl]
                return 0
            lax.fori_loop(0, CH, addrow, 0)

        def gwait(i, ab, cb, sg, sc):
            pltpu.make_async_copy(
                a_hbm.at[srcall.at[pl.ds(i * CH, CH)]], ab, sg).wait()
            pltpu.make_async_copy(
                c_hbm.at[pl.ds(tbase + i * CH, CH), :], cb, sc).wait()

        def wwait(i, mb, sw):
            pltpu.make_async_copy(
                mb, m_hbm.at[pl.ds(tbase + i * CH, CH), :], sw).wait()

        issue(0, ab0, cb0, sg0, sc0)
        issue(1, ab1, cb1, sg1, sc1)

        def pair(pi, _):
            i0 = 2 * pi
            gwait(i0, ab0, cb0, sg0, sc0)

            @pl.when(pi > 0)
            def _():
                wwait(i0 - 2, mb0, sw0)

            adds(ab0, cb0, mb0)
            pltpu.async_copy(mb0, m_hbm.at[pl.ds(tbase + i0 * CH, CH), :], sw0)

            @pl.when(i0 + 2 < NCH)
            def _():
                issue(i0 + 2, ab0, cb0, sg0, sc0)

            i1 = i0 + 1
            gwait(i1, ab1, cb1, sg1, sc1)

            @pl.when(pi > 0)
            def _():
                wwait(i1 - 2, mb1, sw1)

            adds(ab1, cb1, mb1)
            pltpu.async_copy(mb1, m_hbm.at[pl.ds(tbase + i1 * CH, CH), :], sw1)

            @pl.when(i1 + 2 < NCH)
            def _():
                issue(i1 + 2, ab1, cb1, sg1, sc1)

            return 0

        lax.fori_loop(0, NCH // 2, pair, 0)

        last = NCH - 1
        gwait(last, ab0, cb0, sg0, sc0)
        wwait(last - 2, mb0, sw0)
        adds(ab0, cb0, mb0)
        pltpu.async_copy(mb0, m_hbm.at[pl.ds(tbase + last * CH, CH), :], sw0)
        wwait(last - 1, mb1, sw1)
        wwait(last, mb0, sw0)

    return k(A, C, src)


def _segment_pass(M, dst):
    mesh = plsc.VectorSubcoreMesh(core_axis_name="c", subcore_axis_name="s")

    @functools.partial(
        pl.kernel,
        out_type=[
            jax.ShapeDtypeStruct((NBK, SEGK + 1, D), jnp.float32),
            jax.ShapeDtypeStruct((NBK, SEGK + 1, D), jnp.float32),
            jax.ShapeDtypeStruct((NBK, SEGK + 1, D), jnp.float32),
            jax.ShapeDtypeStruct((NBK, SEGK + 1, 16), jnp.float32),
        ],
        mesh=mesh,
        compiler_params=_SC_PARAMS,
        scratch_types=[
            pltpu.VMEM((SEGK + 1, D), jnp.float32),
            pltpu.VMEM((SEGK + 1, D), jnp.float32),
            pltpu.VMEM((SEGK + 1, D), jnp.float32),
            pltpu.VMEM((SEGK + 1, 16), jnp.float32),
            pltpu.VMEM((SC_CH,), jnp.int32),
            pltpu.VMEM((SC_CH,), jnp.int32),
            pltpu.VMEM((QL // 8, 128), jnp.int32),
            pltpu.VMEM((2, RB), jnp.int32),
            pltpu.VMEM((2, RB), jnp.int32),
            pltpu.VMEM((2 * RB, D), jnp.float32),
            pltpu.SemaphoreType.DMA,
            pltpu.SemaphoreType.DMA,
            pltpu.SemaphoreType.DMA,
            pltpu.SemaphoreType.DMA,
        ],
    )
    def k(m_hbm, dst_hbm, sum_hbm, max_hbm, min_hbm, deg_hbm,
          sacc, xacc, nacc, dacc, dbuf0, dbuf1, pq2, stg, dls, mbuf,
          semg0, semg1, semd0, semd1):
        t = lax.axis_index("s") * 2 + lax.axis_index("c")
        iota = lax.broadcasted_iota(jnp.int32, (16,), 0)
        one0 = jnp.where(iota == 0, 1.0, 0.0)
        zero16 = jnp.zeros((16,), jnp.float32)
        zi = jnp.zeros((16,), jnp.int32)

        def phase(p):
            bkt = 2 * t + p
            lo = bkt * SEGK
            hi = lo + SEGK

            def init(i, _):
                for f in range(D // 16):
                    sl = pl.ds(f * 16, 16)
                    sacc[i, sl] = zero16
                    xacc[i, sl] = zero16 + FMIN
                    nacc[i, sl] = zero16 + FMAX
                dacc[i, pl.ds(0, 16)] = zero16
                return 0

            lax.fori_loop(0, SEGK + 1, init, 0)

            def form_issue(rc, prv, cnt):
                slot = rc & 1
                backlog = cnt - prv
                for r in range(RB // 16):
                    valid = backlog > r
                    pos = prv + r
                    rowr = (pos >> 3) & (QL // 8 - 1)
                    colr = iota * 8 + (pos & 7)
                    ent = plsc.load_gather(pq2, [rowr, colr])
                    dlv = jnp.where(valid, ent & 255, SEGK)
                    ev = jnp.where(valid, ent >> 8, 0)
                    stg[slot, pl.ds(r * 16, 16)] = ev
                    dls[slot, pl.ds(r * 16, 16)] = dlv

                @pl.when(slot == 0)
                def _():
                    pltpu.async_copy(
                        m_hbm.at[stg.at[0]], mbuf.at[pl.ds(0, RB), :], semg0)

                @pl.when(slot == 1)
                def _():
                    pltpu.async_copy(
                        m_hbm.at[stg.at[1]], mbuf.at[pl.ds(RB, RB), :], semg1)

                return prv + jnp.minimum(backlog, RB // 16)

            def process(rc):
                slot = rc & 1

                @pl.when(slot == 0)
                def _():
                    pltpu.make_async_copy(
                        m_hbm.at[stg.at[0]], mbuf.at[pl.ds(0, RB), :],
                        semg0).wait()

                @pl.when(slot == 1)
                def _():
                    pltpu.make_async_copy(
                        m_hbm.at[stg.at[1]], mbuf.at[pl.ds(RB, RB), :],
                        semg1).wait()

                def rmw(r, _):
                    dlv = dls[slot, pl.ds(r * 16, 16)]
                    for j in range(16):
                        dlj = jnp.sum(jnp.where(iota == j, dlv, 0))
                        row = slot * RB + r * 16 + j
                        for f in range(D // 16):
                            sl = pl.ds(f * 16, 16)
                            mrow = mbuf[row, sl]
                            sacc[dlj, sl] = sacc[dlj, sl] + mrow
                            xacc[dlj, sl] = jnp.maximum(xacc[dlj, sl], mrow)
                            nacc[dlj, sl] = jnp.minimum(nacc[dlj, sl], mrow)
                        dacc[dlj, pl.ds(0, 16)] = (
                            dacc[dlj, pl.ds(0, 16)] + one0)
                    return 0

                lax.fori_loop(0, RB // 16, rmw, 0)

            def step(c):
                prv, cnt, rc = c
                prv = form_issue(rc, prv, cnt)

                @pl.when(rc > 0)
                def _():
                    process(rc - 1)

                return (prv, cnt, rc + 1)

            def scan_chunk(dbuf, oi, carry):
                def vec(ii, c):
                    prv, cnt, rc = c
                    kv = dbuf[pl.ds(ii * 16, 16)]
                    m = (kv >= lo) & (kv < hi)
                    ent = ((oi * SC_CH + ii * 16 + iota) << 8) | (kv - lo)
                    rows = (cnt >> 3) & (QL // 8 - 1)
                    cols = iota * 8 + (cnt & 7)
                    plsc.store_scatter(pq2, [rows, cols], ent, mask=m)
                    return (prv, cnt + jnp.where(m, 1, 0), rc)

                carry = lax.fori_loop(0, SC_CH // 16, vec, carry, unroll=4)
                return lax.while_loop(
                    lambda c: jnp.sum(c[1] - c[0]) >= RB, step, carry)

            def pair(i, carry):
                pltpu.make_async_copy(
                    dst_hbm.at[pl.ds((2 * i) * SC_CH, SC_CH)], dbuf0,
                    semd0).wait()
                pltpu.async_copy(
                    dst_hbm.at[pl.ds((2 * i + 1) * SC_CH, SC_CH)], dbuf1,
                    semd1)
                carry = scan_chunk(dbuf0, 2 * i, carry)
                pltpu.make_async_copy(
                    dst_hbm.at[pl.ds((2 * i + 1) * SC_CH, SC_CH)], dbuf1,
                    semd1).wait()

                @pl.when(2 * i + 2 < N_OUT)
                def _():
                    pltpu.async_copy(
                        dst_hbm.at[pl.ds((2 * i + 2) * SC_CH, SC_CH)], dbuf0,
                        semd0)

                return scan_chunk(dbuf1, 2 * i + 1, carry)

            pltpu.async_copy(dst_hbm.at[pl.ds(0, SC_CH)], dbuf0, semd0)
            carry = lax.fori_loop(0, N_OUT // 2, pair, (zi, zi, 0))

            carry = lax.while_loop(
                lambda c: jnp.sum(c[1] - c[0]) > 0, step, carry)
            prv, cnt, rc = carry

            @pl.when(rc > 0)
            def _():
                process(rc - 1)

            pltpu.sync_copy(sacc, sum_hbm.at[bkt])
            pltpu.sync_copy(xacc, max_hbm.at[bkt])
            pltpu.sync_copy(nacc, min_hbm.at[bkt])
            pltpu.sync_copy(dacc, deg_hbm.at[bkt])

        phase(0)
        phase(1)

    return k(M, dst)


def _post_body(h_ref, bmat_ref, sum_ref, mx_ref, mn_ref, deg_ref, sn_ref,
               wp_ref, bp_ref, out_ref, stat_ref):
    i = pl.program_id(0)
    deg = deg_ref[...]
    pos = deg > 0.0
    invd = 1.0 / jnp.maximum(deg, 1.0)
    bmat = bmat_ref[...]
    mask = jnp.where(pos, 1.0, 0.0)
    mean = sum_ref[...] * invd + bmat * mask
    mx = jnp.where(pos, mx_ref[...] + bmat, 0.0)
    mn = jnp.where(pos, mn_ref[...] + bmat, 0.0)
    o = (
        jnp.dot(h_ref[...], wp_ref[0:D, :], preferred_element_type=jnp.float32)
        + jnp.dot(mean, wp_ref[D:2 * D, :], preferred_element_type=jnp.float32)
        + jnp.dot(mx, wp_ref[2 * D:3 * D, :], preferred_element_type=jnp.float32)
        + jnp.dot(mn, wp_ref[3 * D:4 * D, :], preferred_element_type=jnp.float32)
        + bp_ref[...]
    )
    o = o * sn_ref[...]
    out_ref[...] = o

    @pl.when(i == 0)
    def _():
        stat_ref[...] = jnp.zeros_like(stat_ref)

    s = jnp.sum(o, axis=0, keepdims=True)
    sq = jnp.sum(o * o, axis=0, keepdims=True)
    lane = lax.broadcasted_iota(jnp.int32, (8, D), 0)
    upd = jnp.where(lane == 0, s, 0.0) + jnp.where(lane == 1, sq, 0.0)
    stat_ref[...] = stat_ref[...] + upd


def _compute_post(h, bmat, sum_f, mx_f, mn_f, deg, snorm, W_post, b_post2):
    blk = 1000
    grid = N_NODES // blk
    return pl.pallas_call(
        _post_body,
        grid=(grid,),
        in_specs=[
            pl.BlockSpec((blk, D), lambda i: (i, 0)),
            pl.BlockSpec((blk, D), lambda i: (i, 0)),
            pl.BlockSpec((blk, D), lambda i: (i, 0)),
            pl.BlockSpec((blk, D), lambda i: (i, 0)),
            pl.BlockSpec((blk, D), lambda i: (i, 0)),
            pl.BlockSpec((blk, 1), lambda i: (i, 0)),
            pl.BlockSpec((blk, 1), lambda i: (i, 0)),
            pl.BlockSpec((4 * D, D), lambda i: (0, 0)),
            pl.BlockSpec((1, D), lambda i: (0, 0)),
        ],
        out_specs=[
            pl.BlockSpec((blk, D), lambda i: (i, 0)),
            pl.BlockSpec((8, D), lambda i: (0, 0)),
        ],
        out_shape=[
            jax.ShapeDtypeStruct((N_NODES, D), jnp.float32),
            jax.ShapeDtypeStruct((8, D), jnp.float32),
        ],
    )(h, bmat, sum_f, mx_f, mn_f, deg, snorm, W_post, b_post2)


def _bn_body(o_ref, stat_ref, g_ref, b_ref, out_ref):
    st = stat_ref[...]
    mu = st[0:1, :] / N_NODES
    var = st[1:2, :] / N_NODES - mu * mu
    inv = lax.rsqrt(var + EPS)
    out_ref[...] = (o_ref[...] - mu) * inv * g_ref[...] + b_ref[...]


def _compute_bn(out_pre, stats, gamma2, beta2):
    blk = 1000
    grid = N_NODES // blk
    return pl.pallas_call(
        _bn_body,
        grid=(grid,),
        in_specs=[
            pl.BlockSpec((blk, D), lambda i: (i, 0)),
            pl.BlockSpec((8, D), lambda i: (0, 0)),
            pl.BlockSpec((1, D), lambda i: (0, 0)),
            pl.BlockSpec((1, D), lambda i: (0, 0)),
        ],
        out_specs=pl.BlockSpec((blk, D), lambda i: (i, 0)),
        out_shape=jax.ShapeDtypeStruct((N_NODES, D), jnp.float32),
    )(out_pre, stats, gamma2, beta2)


def kernel(h, edge_index, e, snorm_n, eig, W_pre, b_pre, W_post, b_post,
           bn_gamma, bn_beta):
    src = edge_index[0]
    dst = edge_index[1]
    b_pre2 = b_pre.reshape(1, D)
    b_post2 = b_post.reshape(1, D)
    gamma2 = bn_gamma.reshape(1, D)
    beta2 = bn_beta.reshape(1, D)

    A, B = _compute_ab(h, W_pre)
    C = _compute_c(e, W_pre, b_pre2)
    M = _m_builder(A, C, src)
    sum3, mx3, mn3, deg2 = _segment_pass(M, dst)

    sum_f = sum3[:, :SEGK, :].reshape(NPADK, D)[:N_NODES]
    mx_f = mx3[:, :SEGK, :].reshape(NPADK, D)[:N_NODES]
    mn_f = mn3[:, :SEGK, :].reshape(NPADK, D)[:N_NODES]
    deg = deg2[:, :SEGK, 0].reshape(NPADK)[:N_NODES].reshape(N_NODES, 1)

    out_pre, stats = _compute_post(
        h, B, sum_f, mx_f, mn_f, deg, snorm_n, W_post, b_post2)
    return _compute_bn(out_pre, stats, gamma2, beta2)
